# baseline (speedup 1.0000x reference)
import functools

import jax
import jax.numpy as jnp
from jax import lax
from jax.experimental import pallas as pl
from jax.experimental.pallas import tpu as pltpu
from jax.experimental.pallas import tpu_sc as plsc

_TOPK = 8
_E = 64
_T = 16384
_NC = 2
_NS = 16
_L = 16
_NW = _NC * _NS
_TPW = _T // _NW
_BPW = _TPW // 128
_SLAB = _TPW * _E


def _sigmoid(x):
    return 1.0 / (1.0 + jnp.exp(-x))


def _merge_top16(ka, va, kb, vb):
    gt = ka > kb
    eq = ka == kb
    km = jnp.maximum(ka, kb)
    vm = jnp.where(gt, va, vb)
    vm = jnp.where(eq, jnp.minimum(va, vb), vm)
    return km, vm


def _routing_body(logits_hbm, bias_hbm, idx_hbm, w_hbm,
                  slab, skew, bias_v, idx_tm, w_tm, idx_st, w_st, dsem, osem):
    wid = lax.axis_index("s") * _NC + lax.axis_index("c")

    copies = []
    for et in range(_E // 8):
        copies.append(pltpu.async_copy(
            logits_hbm.at[pl.ds(et * (_T * 8) + wid * (_BPW * 1024),
                                _BPW * 1024)],
            slab.at[pl.ds(et * (_BPW * 1024), _BPW * 1024)], dsem))
    pltpu.sync_copy(bias_hbm, bias_v)
    for c in copies:
        c.wait()

    lane = lax.iota(jnp.int32, _L)
    low8 = lane < _TOPK
    bias_r = [bias_v[pl.ds(j * _L, _L)] for j in range(_E // _L)]
    vids = [lane + j * _L for j in range(_E // _L)]

    def relayout(i):
        r = i >> 3
        ec = jnp.bitwise_and(i, 7) * 8
        tv = r * _L + lane
        tv64 = tv * _E
        srow = ((r >> 3) << 10) + ((r & 7) << 4)
        tvm = tv + ec
        for k in range(8):
            src = ec * (_BPW * 128) + k * 128 + srow
            x = slab[pl.ds(src, _L)]
            addr = tv64 + jnp.bitwise_and(tvm + k, _E - 1)
            plsc.store_scatter(skew, [addr], x)

    plsc.parallel_loop(0, (_TPW // _L) * 8, 1, unroll=4)(relayout)

    def _rev(a):
        return lax.rev(a, (0,))

    def _sort2(k, v):
        return lax.sort((k, v), dimension=0, num_keys=1)

    def body(t):
        t64v = jnp.full((_L,), t * _E, dtype=jnp.int32)
        lt = lane + t
        sorted_kv = []
        for j in range(_E // _L):
            gaddr = t64v + jnp.bitwise_and(lt + j * _L, _E - 1)
            x = plsc.load_gather(skew, [gaddr])
            k = _sigmoid(x) + bias_r[j]
            sorted_kv.append(_sort2(k, vids[j]))
        k01, v01 = _merge_top16(_rev(sorted_kv[0][0]), _rev(sorted_kv[0][1]),
                                sorted_kv[1][0], sorted_kv[1][1])
        k23, v23 = _merge_top16(_rev(sorted_kv[2][0]), _rev(sorted_kv[2][1]),
                                sorted_kv[3][0], sorted_kv[3][1])
        k01, v01 = _sort2(k01, v01)
        k23, v23 = _sort2(k23, v23)
        kt, vt = _merge_top16(_rev(k01), _rev(v01), k23, v23)
        _, vf = _sort2(kt, vt)
        top_ids = _rev(vf)

        gaddr = t64v + jnp.bitwise_and(top_ids + t, _E - 1)
        xg = plsc.load_gather(skew, [gaddr])
        sg = jnp.where(low8, _sigmoid(xg), 0.0)
        total = jnp.broadcast_to(jnp.sum(sg), (_L,)) + 1e-20
        w = sg / total

        out_pos = t * _TOPK + jnp.bitwise_and(lane + t, _TOPK - 1)
        plsc.store_scatter(idx_tm, [out_pos], top_ids, mask=low8)
        plsc.store_scatter(w_tm, [out_pos], w, mask=low8)

    plsc.parallel_loop(0, _TPW, 1, unroll=4)(body)

    def unstage(r):
        t0 = r * _L
        tv = t0 + lane
        tv8 = tv * _TOPK
        drow = ((r >> 3) << 10) + ((r & 7) << 4)
        for k in range(_TOPK):
            src = tv8 + jnp.bitwise_and(tv + k, _TOPK - 1)
            iv = plsc.load_gather(idx_tm, [src])
            wv = plsc.load_gather(w_tm, [src])
            idx_st[pl.ds(k * 128 + drow, _L)] = iv
            w_st[pl.ds(k * 128 + drow, _L)] = wv

    plsc.parallel_loop(0, _TPW // _L, 1, unroll=4)(unstage)

    oc1 = pltpu.async_copy(
        idx_st, idx_hbm.at[pl.ds(wid * (_BPW * 1024), _BPW * 1024)], osem)
    oc2 = pltpu.async_copy(
        w_st, w_hbm.at[pl.ds(wid * (_BPW * 1024), _BPW * 1024)], osem)
    oc1.wait()
    oc2.wait()


@jax.jit
def kernel(router_logits, e_score_correction_bias):
    routing = functools.partial(
        pl.kernel,
        out_type=(
            jax.ShapeDtypeStruct((_T * _TOPK,), jnp.int32),
            jax.ShapeDtypeStruct((_T * _TOPK,), jnp.float32),
        ),
        mesh=plsc.VectorSubcoreMesh(
            core_axis_name="c", subcore_axis_name="s",
            num_cores=_NC, num_subcores=_NS),
        scratch_types=[
            pltpu.VMEM((_SLAB,), jnp.float32),
            pltpu.VMEM((_SLAB,), jnp.float32),
            pltpu.VMEM((_E,), jnp.float32),
            pltpu.VMEM((_TPW * _TOPK,), jnp.int32),
            pltpu.VMEM((_TPW * _TOPK,), jnp.float32),
            pltpu.VMEM((_BPW * 1024,), jnp.int32),
            pltpu.VMEM((_BPW * 1024,), jnp.float32),
            pltpu.SemaphoreType.DMA,
            pltpu.SemaphoreType.DMA,
        ],
        compiler_params=pltpu.CompilerParams(needs_layout_passes=False),
    )(_routing_body)
    xl = (router_logits.T.reshape(8, 8, 128, 128)
          .transpose(0, 2, 1, 3).reshape(_T * _E))
    idx_flat, w_flat = routing(xl, e_score_correction_bias)
    idx2d = idx_flat.reshape(128, 8, 128).transpose(1, 0, 2).reshape(8, _T).T
    w2d = w_flat.reshape(128, 8, 128).transpose(1, 0, 2).reshape(8, _T).T
    return (idx2d, w2d)


# speedup vs baseline: 1.0362x; 1.0362x over previous
import functools

import jax
import jax.numpy as jnp
from jax import lax
from jax.experimental import pallas as pl
from jax.experimental.pallas import tpu as pltpu
from jax.experimental.pallas import tpu_sc as plsc

_TOPK = 8
_E = 64
_T = 16384
_NC = 2
_NS = 16
_L = 16
_NW = _NC * _NS
_TPW = _T // _NW
_BPW = _TPW // 128
_SLAB = _TPW * _E


def _sigmoid(x):
    return 1.0 / (1.0 + jnp.exp(-x))


def _merge_top16(ka, va, kb, vb):
    gt = ka > kb
    eq = ka == kb
    km = jnp.maximum(ka, kb)
    vm = jnp.where(gt, va, vb)
    vm = jnp.where(eq, jnp.minimum(va, vb), vm)
    return km, vm


def _routing_body(logits_hbm, bias_hbm, idx_hbm, w_hbm,
                  slab, skew, bias_v, idx_tm, w_tm, idx_st, w_st, dsem, osem):
    wid = lax.axis_index("s") * _NC + lax.axis_index("c")

    copies = []
    for et in range(_E // 8):
        copies.append(pltpu.async_copy(
            logits_hbm.at[pl.ds(et * (_T * 8) + wid * (_BPW * 1024),
                                _BPW * 1024)],
            slab.at[pl.ds(et * (_BPW * 1024), _BPW * 1024)], dsem))
    pltpu.sync_copy(bias_hbm, bias_v)
    for c in copies:
        c.wait()

    lane = lax.iota(jnp.int32, _L)
    low8 = lane < _TOPK
    bias_r = [bias_v[pl.ds(j * _L, _L)] for j in range(_E // _L)]
    vids = [lane + j * _L for j in range(_E // _L)]

    def relayout(i):
        r = i >> 3
        ec = jnp.bitwise_and(i, 7) * 8
        tv = r * _L + lane
        tv64 = tv * _E
        srow = ((r >> 3) << 10) + ((r & 7) << 4)
        tvm = tv + ec
        for k in range(8):
            src = ec * (_BPW * 128) + k * 128 + srow
            x = slab[pl.ds(src, _L)]
            addr = tv64 + jnp.bitwise_and(tvm + k, _E - 1)
            plsc.store_scatter(skew, [addr], x)

    plsc.parallel_loop(0, (_TPW // _L) * 8, 1, unroll=4)(relayout)

    def body(t):
        t64v = jnp.full((_L,), t * _E, dtype=jnp.int32)
        lt = lane + t
        sorted_kv = []
        for j in range(_E // _L):
            gaddr = t64v + jnp.bitwise_and(lt + j * _L, _E - 1)
            x = plsc.load_gather(skew, [gaddr])
            k = _sigmoid(x) + bias_r[j]
            sorted_kv.append(
                plsc.sort_key_val(k, vids[j], descending=(j % 2 == 0)))
        k01, v01 = _merge_top16(sorted_kv[0][0], sorted_kv[0][1],
                                sorted_kv[1][0], sorted_kv[1][1])
        k23, v23 = _merge_top16(sorted_kv[2][0], sorted_kv[2][1],
                                sorted_kv[3][0], sorted_kv[3][1])
        k01, v01 = plsc.sort_key_val(k01, v01, descending=True)
        k23, v23 = plsc.sort_key_val(k23, v23, descending=False)
        kt, vt = _merge_top16(k01, v01, k23, v23)
        _, top_ids = plsc.sort_key_val(kt, vt, descending=True)

        gaddr = t64v + jnp.bitwise_and(top_ids + t, _E - 1)
        xg = plsc.load_gather(skew, [gaddr])
        sg = jnp.where(low8, _sigmoid(xg), 0.0)
        total = jnp.broadcast_to(jnp.sum(sg), (_L,)) + 1e-20
        w = sg / total

        out_pos = t * _TOPK + jnp.bitwise_and(lane + t, _TOPK - 1)
        plsc.store_scatter(idx_tm, [out_pos], top_ids, mask=low8)
        plsc.store_scatter(w_tm, [out_pos], w, mask=low8)

    plsc.parallel_loop(0, _TPW, 1, unroll=4)(body)

    def unstage(r):
        t0 = r * _L
        tv = t0 + lane
        tv8 = tv * _TOPK
        drow = ((r >> 3) << 10) + ((r & 7) << 4)
        for k in range(_TOPK):
            src = tv8 + jnp.bitwise_and(tv + k, _TOPK - 1)
            iv = plsc.load_gather(idx_tm, [src])
            wv = plsc.load_gather(w_tm, [src])
            idx_st[pl.ds(k * 128 + drow, _L)] = iv
            w_st[pl.ds(k * 128 + drow, _L)] = wv

    plsc.parallel_loop(0, _TPW // _L, 1, unroll=4)(unstage)

    oc1 = pltpu.async_copy(
        idx_st, idx_hbm.at[pl.ds(wid * (_BPW * 1024), _BPW * 1024)], osem)
    oc2 = pltpu.async_copy(
        w_st, w_hbm.at[pl.ds(wid * (_BPW * 1024), _BPW * 1024)], osem)
    oc1.wait()
    oc2.wait()


@jax.jit
def kernel(router_logits, e_score_correction_bias):
    routing = functools.partial(
        pl.kernel,
        out_type=(
            jax.ShapeDtypeStruct((_T * _TOPK,), jnp.int32),
            jax.ShapeDtypeStruct((_T * _TOPK,), jnp.float32),
        ),
        mesh=plsc.VectorSubcoreMesh(
            core_axis_name="c", subcore_axis_name="s",
            num_cores=_NC, num_subcores=_NS),
        scratch_types=[
            pltpu.VMEM((_SLAB,), jnp.float32),
            pltpu.VMEM((_SLAB,), jnp.float32),
            pltpu.VMEM((_E,), jnp.float32),
            pltpu.VMEM((_TPW * _TOPK,), jnp.int32),
            pltpu.VMEM((_TPW * _TOPK,), jnp.float32),
            pltpu.VMEM((_BPW * 1024,), jnp.int32),
            pltpu.VMEM((_BPW * 1024,), jnp.float32),
            pltpu.SemaphoreType.DMA,
            pltpu.SemaphoreType.DMA,
        ],
        compiler_params=pltpu.CompilerParams(needs_layout_passes=False),
    )(_routing_body)
    xl = (router_logits.T.reshape(8, 8, 128, 128)
          .transpose(0, 2, 1, 3).reshape(_T * _E))
    idx_flat, w_flat = routing(xl, e_score_correction_bias)
    idx2d = idx_flat.reshape(128, 8, 128).transpose(1, 0, 2).reshape(8, _T).T
    w2d = w_flat.reshape(128, 8, 128).transpose(1, 0, 2).reshape(8, _T).T
    return (idx2d, w2d)
